# Initial kernel scaffold; baseline (speedup 1.0000x reference)
import jax
import jax.numpy as jnp
from jax.experimental import pallas as pl


def kernel(src_points, tgt_points, src_normals, tgt_normals, src_feats, tgt_feats, src_v, tgt_v, Wn1, Wn2, Wp1, W_coarse, b_coarse, W_fine, b_fine, alpha):
    raise NotImplementedError("write your pallas kernel here")


# speedup vs baseline: 1.2322x; 1.2322x over previous
import functools

import jax
import jax.numpy as jnp
from jax.experimental import pallas as pl
from jax.experimental.pallas import tpu as pltpu

_STRIDE = 64
_POINT_LIMIT = 64
_NUM_CORR = 256
_D_COARSE = 256
_D_FINE = 64
_NUM_ITER = 100
_FINE_TOPK = 3
_FINE_TOPK_TOTAL = 1000
_CONF = 0.05

_B_BLK = 8


def _ot_fine_body(tgt_f_ref, src_f_ref, tgt_m_ref, src_m_ref, alpha_ref,
                  ms_ref, sel_ref):
    B = _B_BLK
    n = _POINT_LIMIT
    alpha = alpha_ref[0, 0]

    tgt_f = tgt_f_ref[...]
    src_f = src_f_ref[...]
    rm = tgt_m_ref[...]
    cm = src_m_ref[...]

    inv_sqrt_d = 1.0 / (_D_COARSE ** 0.5)
    rows = []
    for b in range(B):
        s_b = jnp.dot(tgt_f[b], src_f[b].T,
                      preferred_element_type=jnp.float32) * inv_sqrt_d
        rows.append(s_b[None])
    s = jnp.concatenate(rows, axis=0)

    top = jnp.concatenate([s, jnp.full((B, n, 1), alpha, jnp.float32)], axis=2)
    padded = jnp.concatenate(
        [top, jnp.full((B, 1, n + 1), alpha, jnp.float32)], axis=1)
    prow = jnp.concatenate([rm, jnp.ones((B, 1), jnp.float32)], axis=1)
    pcol = jnp.concatenate([cm, jnp.ones((B, 1), jnp.float32)], axis=1)
    smask = prow[:, :, None] * pcol[:, None, :]
    padded = jnp.where(smask > 0.0, padded, -1e9)

    nvr = jnp.sum(rm, axis=1)
    nvc = jnp.sum(cm, axis=1)
    norm = -jnp.log(nvr + nvc)
    log_mu = jnp.concatenate(
        [jnp.broadcast_to(norm[:, None], (B, n)), (jnp.log(nvc) + norm)[:, None]],
        axis=1)
    log_mu = jnp.where(prow > 0.0, log_mu, -1e9)
    log_nu = jnp.concatenate(
        [jnp.broadcast_to(norm[:, None], (B, n)), (jnp.log(nvr) + norm)[:, None]],
        axis=1)
    log_nu = jnp.where(pcol > 0.0, log_nu, -1e9)

    def lse(x, axis):
        m = jnp.max(x, axis=axis)
        return jnp.log(jnp.sum(jnp.exp(x - jnp.expand_dims(m, axis)),
                               axis=axis)) + m

    def body(_, carry):
        u, v = carry
        u = log_mu - lse(padded + v[:, None, :], 2)
        v = log_nu - lse(padded + u[:, :, None], 1)
        return (u, v)

    u0 = jnp.zeros((B, n + 1), jnp.float32)
    v0 = jnp.zeros((B, n + 1), jnp.float32)
    u, v = jax.lax.fori_loop(0, _NUM_ITER, body, (u0, v0))

    ms = padded + u[:, :, None] + v[:, None, :] - norm[:, None, None]
    ms_ref[...] = ms

    sc = jnp.exp(ms[:, :n, :n])
    valid = (rm[:, :, None] * cm[:, None, :]) > 0.0
    sc = jnp.where(valid, sc, 0.0)

    def third_largest(x, axis):
        m1 = jnp.max(x, axis=axis, keepdims=True)
        c1 = jnp.sum((x == m1).astype(jnp.float32), axis=axis, keepdims=True)
        x2 = jnp.where(x < m1, x, -1.0)
        m2 = jnp.max(x2, axis=axis, keepdims=True)
        c2 = jnp.sum((x == m2).astype(jnp.float32), axis=axis, keepdims=True)
        m3 = jnp.max(jnp.where(x2 < m2, x2, -1.0), axis=axis, keepdims=True)
        return jnp.where(c1 >= 3.0, m1, jnp.where(c1 + c2 >= 3.0, m2, m3))

    row_mask = sc >= third_largest(sc, 2)
    col_mask = sc >= third_largest(sc, 1)

    corr = row_mask & col_mask & (sc > _CONF) & valid
    sel_ref[...] = jnp.where(corr, sc, 0.0)


def _ot_fine(tgt_ckf, src_ckf, tgt_ckm, src_ckm, alpha):
    b = tgt_ckf.shape[0]
    grid = (b // _B_BLK,)
    n = _POINT_LIMIT
    out = pl.pallas_call(
        _ot_fine_body,
        grid=grid,
        in_specs=[
            pl.BlockSpec((_B_BLK, n, _D_COARSE), lambda i: (i, 0, 0)),
            pl.BlockSpec((_B_BLK, n, _D_COARSE), lambda i: (i, 0, 0)),
            pl.BlockSpec((_B_BLK, n), lambda i: (i, 0)),
            pl.BlockSpec((_B_BLK, n), lambda i: (i, 0)),
            pl.BlockSpec(memory_space=pltpu.SMEM),
        ],
        out_specs=[
            pl.BlockSpec((_B_BLK, n + 1, n + 1), lambda i: (i, 0, 0)),
            pl.BlockSpec((_B_BLK, n, n), lambda i: (i, 0, 0)),
        ],
        out_shape=[
            jax.ShapeDtypeStruct((b, n + 1, n + 1), jnp.float32),
            jax.ShapeDtypeStruct((b, n, n), jnp.float32),
        ],
    )(tgt_ckf, src_ckf, tgt_ckm.astype(jnp.float32),
      src_ckm.astype(jnp.float32), alpha.reshape(1, 1))
    return out


def _backbone(points, normals, feats, v, Wn1, Wn2, Wp1):
    h = jnp.concatenate([points, normals, feats, v], axis=1)
    point_feats = jax.nn.relu(h @ Wp1)
    node_xyz = points[::_STRIDE]
    node_h = h[::_STRIDE]
    node_feats = jax.nn.relu(node_h @ Wn1) @ Wn2
    return node_xyz, node_feats, point_feats


def _point_to_node_partition(points, nodes, point_limit):
    dist2 = (jnp.sum(nodes ** 2, axis=1)[:, None]
             + jnp.sum(points ** 2, axis=1)[None, :]
             - 2.0 * (nodes @ points.T))
    point_to_node = jnp.argmin(dist2, axis=0)
    n_nodes = nodes.shape[0]
    counts = jnp.bincount(point_to_node, length=n_nodes)
    node_masks = counts > 0
    _, knn_indices = jax.lax.top_k(-dist2, point_limit)
    knn_masks = point_to_node[knn_indices] == jnp.arange(n_nodes)[:, None]
    knn_indices = jnp.where(knn_masks, knn_indices, points.shape[0])
    return point_to_node, node_masks, knn_indices, knn_masks


def _coarse_matching(tgt_feats, src_feats, tgt_masks, src_masks, k):
    sim = jnp.exp(tgt_feats @ src_feats.T)
    row = sim / (jnp.sum(sim, axis=1, keepdims=True) + 1e-12)
    col = sim / (jnp.sum(sim, axis=0, keepdims=True) + 1e-12)
    score = row * col
    mask = tgt_masks[:, None] & src_masks[None, :]
    score = jnp.where(mask, score, 0.0)
    vals, idx = jax.lax.top_k(score.reshape(-1), k)
    m = src_feats.shape[0]
    return idx // m, idx % m, vals


def kernel(src_points, tgt_points, src_normals, tgt_normals, src_feats,
           tgt_feats, src_v, tgt_v, Wn1, Wn2, Wp1, W_coarse, b_coarse,
           W_fine, b_fine, alpha):
    src_node_xyz, src_node_feats, src_point_feats = _backbone(
        src_points, src_normals, src_feats, src_v, Wn1, Wn2, Wp1)
    tgt_node_xyz, tgt_node_feats, tgt_point_feats = _backbone(
        tgt_points, tgt_normals, tgt_feats, tgt_v, Wn1, Wn2, Wp1)

    def l2n(x):
        return x / (jnp.linalg.norm(x, axis=1, keepdims=True) + 1e-12)

    src_node_feats = l2n(src_node_feats @ W_coarse + b_coarse)
    tgt_node_feats = l2n(tgt_node_feats @ W_coarse + b_coarse)
    src_point_feats = src_point_feats @ W_fine + b_fine
    tgt_point_feats = tgt_point_feats @ W_fine + b_fine

    _, src_node_masks, src_knn_idx, src_knn_masks = _point_to_node_partition(
        src_points, src_node_xyz, _POINT_LIMIT)
    _, tgt_node_masks, tgt_knn_idx, tgt_knn_masks = _point_to_node_partition(
        tgt_points, tgt_node_xyz, _POINT_LIMIT)

    tgt_ci, src_ci, _ = _coarse_matching(
        tgt_node_feats, src_node_feats, tgt_node_masks, src_node_masks,
        _NUM_CORR)

    src_ckm = src_knn_masks[src_ci]
    tgt_ckm = tgt_knn_masks[tgt_ci]
    src_ckidx = src_knn_idx[src_ci]
    tgt_ckidx = tgt_knn_idx[tgt_ci]

    src_pp_feats = jnp.concatenate(
        [src_point_feats, jnp.zeros((1, _D_COARSE), jnp.float32)], axis=0)
    tgt_pp_feats = jnp.concatenate(
        [tgt_point_feats, jnp.zeros((1, _D_COARSE), jnp.float32)], axis=0)
    src_ckf = src_pp_feats[src_ckidx]
    tgt_ckf = tgt_pp_feats[tgt_ckidx]

    ms, sel = _ot_fine(tgt_ckf, src_ckf, tgt_ckm, src_ckm, alpha)

    b, n, m = sel.shape
    vals, idx = jax.lax.top_k(sel.reshape(-1), _FINE_TOPK_TOTAL)
    bi = idx // (n * m)
    rem = idx % (n * m)

    src_padded_points = jnp.concatenate(
        [src_points, jnp.zeros((1, 3), src_points.dtype)], axis=0)
    tgt_padded_points = jnp.concatenate(
        [tgt_points, jnp.zeros((1, 3), tgt_points.dtype)], axis=0)
    tgt_corr_points = tgt_padded_points[tgt_ckidx[bi, rem // m]]
    src_corr_points = src_padded_points[src_ckidx[bi, rem % m]]
    return ms, tgt_corr_points, src_corr_points, vals


# speedup vs baseline: 1.8428x; 1.4956x over previous
import functools

import jax
import jax.numpy as jnp
from jax.experimental import pallas as pl
from jax.experimental.pallas import tpu as pltpu

_STRIDE = 64
_POINT_LIMIT = 64
_NUM_CORR = 256
_D_COARSE = 256
_D_FINE = 64
_NUM_ITER = 100
_FINE_TOPK = 3
_FINE_TOPK_TOTAL = 1000
_CONF = 0.05

_B_SIM = 8


def _sim_body(tgt_f_ref, src_f_ref, s_ref):
    inv_sqrt_d = 1.0 / (_D_COARSE ** 0.5)
    tgt_f = tgt_f_ref[...]
    src_f = src_f_ref[...]
    rows = []
    for b in range(_B_SIM):
        s_b = jnp.dot(tgt_f[b], src_f[b].T,
                      preferred_element_type=jnp.float32) * inv_sqrt_d
        rows.append(s_b[None])
    s_ref[...] = jnp.concatenate(rows, axis=0)


def _ot_fine_body(sT_ref, rmT_ref, cmT_ref, alpha_ref,
                  msT_ref, candv_ref, candi_ref, win_ref):
    n = _POINT_LIMIT
    B = _NUM_CORR
    alpha = alpha_ref[0, 0]

    sT = sT_ref[...]
    rmT = rmT_ref[...]
    cmT = cmT_ref[...]

    arow = jnp.full((n, 1, B), alpha, jnp.float32)
    acol = jnp.full((1, n + 1, B), alpha, jnp.float32)
    padded = jnp.concatenate(
        [jnp.concatenate([sT, arow], axis=1), acol], axis=0)
    prow = jnp.concatenate([rmT, jnp.ones((1, B), jnp.float32)], axis=0)
    pcol = jnp.concatenate([cmT, jnp.ones((1, B), jnp.float32)], axis=0)
    smask = prow[:, None, :] * pcol[None, :, :]
    padded = jnp.where(smask > 0.0, padded, -1e9)

    nvr = jnp.sum(rmT, axis=0, keepdims=True)
    nvc = jnp.sum(cmT, axis=0, keepdims=True)
    norm = -jnp.log(nvr + nvc)
    log_mu = jnp.concatenate(
        [jnp.broadcast_to(norm, (n, B)), jnp.log(nvc) + norm], axis=0)
    log_mu = jnp.where(prow > 0.0, log_mu, -1e9)
    log_nu = jnp.concatenate(
        [jnp.broadcast_to(norm, (n, B)), jnp.log(nvr) + norm], axis=0)
    log_nu = jnp.where(pcol > 0.0, log_nu, -1e9)

    def lse(x, axis):
        m = jnp.max(x, axis=axis)
        return jnp.log(jnp.sum(jnp.exp(x - jnp.expand_dims(m, axis)),
                               axis=axis)) + m

    def body(_, carry):
        u, v = carry
        u = log_mu - lse(padded + v[None, :, :], 1)
        v = log_nu - lse(padded + u[:, None, :], 0)
        return (u, v)

    u0 = jnp.zeros((n + 1, B), jnp.float32)
    v0 = jnp.zeros((n + 1, B), jnp.float32)
    u, v = jax.lax.fori_loop(0, _NUM_ITER, body, (u0, v0))

    msT = padded + u[:, None, :] + v[None, :, :] - norm[None, :, :]
    msT_ref[...] = msT

    sc = jnp.exp(msT[:n, :n, :])
    valid = (rmT[:, None, :] * cmT[None, :, :]) > 0.0
    sc = jnp.where(valid, sc, 0.0)

    def third_largest(x, axis):
        m1 = jnp.max(x, axis=axis, keepdims=True)
        c1 = jnp.sum((x == m1).astype(jnp.float32), axis=axis, keepdims=True)
        x2 = jnp.where(x < m1, x, -1.0)
        m2 = jnp.max(x2, axis=axis, keepdims=True)
        c2 = jnp.sum((x == m2).astype(jnp.float32), axis=axis, keepdims=True)
        m3 = jnp.max(jnp.where(x2 < m2, x2, -1.0), axis=axis, keepdims=True)
        return jnp.where(c1 >= 3.0, m1, jnp.where(c1 + c2 >= 3.0, m2, m3))

    row_mask = sc >= third_largest(sc, 1)
    col_mask = sc >= third_largest(sc, 0)
    corr = row_mask & col_mask & (sc > _CONF) & valid
    sel = jnp.where(corr, sc, 0.0)

    win_ref[...] = sel[:32, :, 0:1]

    iota_j = jax.lax.broadcasted_iota(jnp.int32, (n, n, B), 1)
    iota_i = jax.lax.broadcasted_iota(jnp.int32, (n, n, B), 0)
    iota_b = jax.lax.broadcasted_iota(jnp.int32, (n, n, B), 2)
    flat_base = (iota_b * n + iota_i) * n

    cur = sel
    cvs, cis = [], []
    for _ in range(_FINE_TOPK):
        vk = jnp.max(cur, axis=1, keepdims=True)
        jk = jnp.min(jnp.where(cur == vk, iota_j, n), axis=1,
                     keepdims=True)
        cvs.append(jnp.where(vk > 0.0, vk, -1.0))
        cis.append(jnp.min(jnp.where(cur == vk, flat_base + iota_j,
                                     jnp.int32(2 ** 30)), axis=1,
                           keepdims=True))
        cur = jnp.where(iota_j == jk, -1.0, cur)
    candv_ref[...] = jnp.concatenate(cvs, axis=1)
    candi_ref[...] = jnp.concatenate(cis, axis=1)


def _ot_fine(tgt_ckf, src_ckf, tgt_ckm, src_ckm, alpha):
    b = tgt_ckf.shape[0]
    n = _POINT_LIMIT
    s = pl.pallas_call(
        _sim_body,
        grid=(b // _B_SIM,),
        in_specs=[
            pl.BlockSpec((_B_SIM, n, _D_COARSE), lambda i: (i, 0, 0)),
            pl.BlockSpec((_B_SIM, n, _D_COARSE), lambda i: (i, 0, 0)),
        ],
        out_specs=pl.BlockSpec((_B_SIM, n, n), lambda i: (i, 0, 0)),
        out_shape=jax.ShapeDtypeStruct((b, n, n), jnp.float32),
    )(tgt_ckf, src_ckf)

    sT = jnp.transpose(s, (1, 2, 0))
    rmT = jnp.transpose(tgt_ckm.astype(jnp.float32))
    cmT = jnp.transpose(src_ckm.astype(jnp.float32))

    msT, candv, candi, win = pl.pallas_call(
        _ot_fine_body,
        in_specs=[
            pl.BlockSpec((n, n, b), lambda: (0, 0, 0)),
            pl.BlockSpec((n, b), lambda: (0, 0)),
            pl.BlockSpec((n, b), lambda: (0, 0)),
            pl.BlockSpec(memory_space=pltpu.SMEM),
        ],
        out_specs=[
            pl.BlockSpec((n + 1, n + 1, b), lambda: (0, 0, 0)),
            pl.BlockSpec((n, _FINE_TOPK, b), lambda: (0, 0, 0)),
            pl.BlockSpec((n, _FINE_TOPK, b), lambda: (0, 0, 0)),
            pl.BlockSpec((32, n, 1), lambda: (0, 0, 0)),
        ],
        out_shape=[
            jax.ShapeDtypeStruct((n + 1, n + 1, b), jnp.float32),
            jax.ShapeDtypeStruct((n, _FINE_TOPK, b), jnp.float32),
            jax.ShapeDtypeStruct((n, _FINE_TOPK, b), jnp.int32),
            jax.ShapeDtypeStruct((32, n, 1), jnp.float32),
        ],
    )(sT, rmT, cmT, alpha.reshape(1, 1))

    ms = jnp.transpose(msT, (2, 0, 1))
    return ms, candv, candi, win


def _fine_topk(candv, candi, win):
    n = _POINT_LIMIT
    cand_v = candv.reshape(-1)
    cand_i = candi.reshape(-1)
    w = win.reshape(-1)
    zkey = jnp.where(w == 0.0, -jnp.arange(w.shape[0], dtype=jnp.float32),
                     -3e9)
    _, zidx = jax.lax.top_k(zkey, _FINE_TOPK_TOTAL)
    allv = jnp.concatenate([cand_v, jnp.zeros((_FINE_TOPK_TOTAL,), jnp.float32)])
    alli = jnp.concatenate([cand_i, zidx.astype(jnp.int32)])
    vals, pos = jax.lax.top_k(allv, _FINE_TOPK_TOTAL)
    idx = alli[pos]
    return vals, idx


def _backbone(points, normals, feats, v, Wn1, Wn2, Wp1):
    h = jnp.concatenate([points, normals, feats, v], axis=1)
    point_feats = jax.nn.relu(h @ Wp1)
    node_xyz = points[::_STRIDE]
    node_h = h[::_STRIDE]
    node_feats = jax.nn.relu(node_h @ Wn1) @ Wn2
    return node_xyz, node_feats, point_feats


def _point_to_node_partition(points, nodes, point_limit):
    dist2 = (jnp.sum(nodes ** 2, axis=1)[:, None]
             + jnp.sum(points ** 2, axis=1)[None, :]
             - 2.0 * (nodes @ points.T))
    point_to_node = jnp.argmin(dist2, axis=0)
    n_nodes = nodes.shape[0]
    counts = jnp.bincount(point_to_node, length=n_nodes)
    node_masks = counts > 0
    _, knn_indices = jax.lax.top_k(-dist2, point_limit)
    knn_masks = point_to_node[knn_indices] == jnp.arange(n_nodes)[:, None]
    knn_indices = jnp.where(knn_masks, knn_indices, points.shape[0])
    return point_to_node, node_masks, knn_indices, knn_masks


def _coarse_matching(tgt_feats, src_feats, tgt_masks, src_masks, k):
    sim = jnp.exp(tgt_feats @ src_feats.T)
    row = sim / (jnp.sum(sim, axis=1, keepdims=True) + 1e-12)
    col = sim / (jnp.sum(sim, axis=0, keepdims=True) + 1e-12)
    score = row * col
    mask = tgt_masks[:, None] & src_masks[None, :]
    score = jnp.where(mask, score, 0.0)
    vals, idx = jax.lax.top_k(score.reshape(-1), k)
    m = src_feats.shape[0]
    return idx // m, idx % m, vals


def kernel(src_points, tgt_points, src_normals, tgt_normals, src_feats,
           tgt_feats, src_v, tgt_v, Wn1, Wn2, Wp1, W_coarse, b_coarse,
           W_fine, b_fine, alpha):
    src_node_xyz, src_node_feats, src_point_feats = _backbone(
        src_points, src_normals, src_feats, src_v, Wn1, Wn2, Wp1)
    tgt_node_xyz, tgt_node_feats, tgt_point_feats = _backbone(
        tgt_points, tgt_normals, tgt_feats, tgt_v, Wn1, Wn2, Wp1)

    def l2n(x):
        return x / (jnp.linalg.norm(x, axis=1, keepdims=True) + 1e-12)

    src_node_feats = l2n(src_node_feats @ W_coarse + b_coarse)
    tgt_node_feats = l2n(tgt_node_feats @ W_coarse + b_coarse)
    src_point_feats = src_point_feats @ W_fine + b_fine
    tgt_point_feats = tgt_point_feats @ W_fine + b_fine

    _, src_node_masks, src_knn_idx, src_knn_masks = _point_to_node_partition(
        src_points, src_node_xyz, _POINT_LIMIT)
    _, tgt_node_masks, tgt_knn_idx, tgt_knn_masks = _point_to_node_partition(
        tgt_points, tgt_node_xyz, _POINT_LIMIT)

    tgt_ci, src_ci, _ = _coarse_matching(
        tgt_node_feats, src_node_feats, tgt_node_masks, src_node_masks,
        _NUM_CORR)

    src_ckm = src_knn_masks[src_ci]
    tgt_ckm = tgt_knn_masks[tgt_ci]
    src_ckidx = src_knn_idx[src_ci]
    tgt_ckidx = tgt_knn_idx[tgt_ci]

    src_pp_feats = jnp.concatenate(
        [src_point_feats, jnp.zeros((1, _D_COARSE), jnp.float32)], axis=0)
    tgt_pp_feats = jnp.concatenate(
        [tgt_point_feats, jnp.zeros((1, _D_COARSE), jnp.float32)], axis=0)
    src_ckf = src_pp_feats[src_ckidx]
    tgt_ckf = tgt_pp_feats[tgt_ckidx]

    ms, candv, candi, win = _ot_fine(tgt_ckf, src_ckf, tgt_ckm, src_ckm, alpha)
    vals, idx = _fine_topk(candv, candi, win)

    n = m = _POINT_LIMIT
    bi = idx // (n * m)
    rem = idx % (n * m)

    src_padded_points = jnp.concatenate(
        [src_points, jnp.zeros((1, 3), src_points.dtype)], axis=0)
    tgt_padded_points = jnp.concatenate(
        [tgt_points, jnp.zeros((1, 3), tgt_points.dtype)], axis=0)
    tgt_corr_points = tgt_padded_points[tgt_ckidx[bi, rem // m]]
    src_corr_points = src_padded_points[src_ckidx[bi, rem % m]]
    return ms, tgt_corr_points, src_corr_points, vals


# speedup vs baseline: 4.4686x; 2.4249x over previous
import functools

import jax
import jax.numpy as jnp
from jax import lax
from jax.experimental import pallas as pl
from jax.experimental.pallas import tpu as pltpu
from jax.experimental.pallas import tpu_sc as plsc

_STRIDE = 64
_POINT_LIMIT = 64
_NUM_CORR = 256
_D_COARSE = 256
_D_FINE = 64
_NUM_ITER = 100
_FINE_TOPK = 3
_FINE_TOPK_TOTAL = 1000
_CONF = 0.05

_B_SIM = 8


def _sim_body(tgt_f_ref, src_f_ref, s_ref):
    inv_sqrt_d = 1.0 / (_D_COARSE ** 0.5)
    tgt_f = tgt_f_ref[...]
    src_f = src_f_ref[...]
    rows = []
    for b in range(_B_SIM):
        s_b = jnp.dot(tgt_f[b], src_f[b].T,
                      preferred_element_type=jnp.float32) * inv_sqrt_d
        rows.append(s_b[None])
    s_ref[...] = jnp.concatenate(rows, axis=0)


def _ot_fine_body(sT_ref, rmT_ref, cmT_ref, alpha_ref,
                  msT_ref, candv_ref, candi_ref, win_ref):
    n = _POINT_LIMIT
    B = _NUM_CORR
    alpha = alpha_ref[0, 0]

    sT = sT_ref[...]
    rmT = rmT_ref[...]
    cmT = cmT_ref[...]

    arow = jnp.full((n, 1, B), alpha, jnp.float32)
    acol = jnp.full((1, n + 1, B), alpha, jnp.float32)
    padded = jnp.concatenate(
        [jnp.concatenate([sT, arow], axis=1), acol], axis=0)
    prow = jnp.concatenate([rmT, jnp.ones((1, B), jnp.float32)], axis=0)
    pcol = jnp.concatenate([cmT, jnp.ones((1, B), jnp.float32)], axis=0)
    smask = prow[:, None, :] * pcol[None, :, :]
    padded = jnp.where(smask > 0.0, padded, -1e9)

    nvr = jnp.sum(rmT, axis=0, keepdims=True)
    nvc = jnp.sum(cmT, axis=0, keepdims=True)
    norm = -jnp.log(nvr + nvc)
    log_mu = jnp.concatenate(
        [jnp.broadcast_to(norm, (n, B)), jnp.log(nvc) + norm], axis=0)
    log_mu = jnp.where(prow > 0.0, log_mu, -1e9)
    log_nu = jnp.concatenate(
        [jnp.broadcast_to(norm, (n, B)), jnp.log(nvr) + norm], axis=0)
    log_nu = jnp.where(pcol > 0.0, log_nu, -1e9)

    def lse(x, axis):
        m = jnp.max(x, axis=axis)
        return jnp.log(jnp.sum(jnp.exp(x - jnp.expand_dims(m, axis)),
                               axis=axis)) + m

    def body(_, carry):
        u, v = carry
        u = log_mu - lse(padded + v[None, :, :], 1)
        v = log_nu - lse(padded + u[:, None, :], 0)
        return (u, v)

    u0 = jnp.zeros((n + 1, B), jnp.float32)
    v0 = jnp.zeros((n + 1, B), jnp.float32)
    u, v = jax.lax.fori_loop(0, _NUM_ITER, body, (u0, v0))

    msT = padded + u[:, None, :] + v[None, :, :] - norm[None, :, :]
    msT_ref[...] = msT

    sc = jnp.exp(msT[:n, :n, :])
    valid = (rmT[:, None, :] * cmT[None, :, :]) > 0.0
    sc = jnp.where(valid, sc, 0.0)

    def third_largest(x, axis):
        m1 = jnp.max(x, axis=axis, keepdims=True)
        c1 = jnp.sum((x == m1).astype(jnp.float32), axis=axis, keepdims=True)
        x2 = jnp.where(x < m1, x, -1.0)
        m2 = jnp.max(x2, axis=axis, keepdims=True)
        c2 = jnp.sum((x == m2).astype(jnp.float32), axis=axis, keepdims=True)
        m3 = jnp.max(jnp.where(x2 < m2, x2, -1.0), axis=axis, keepdims=True)
        return jnp.where(c1 >= 3.0, m1, jnp.where(c1 + c2 >= 3.0, m2, m3))

    row_mask = sc >= third_largest(sc, 1)
    col_mask = sc >= third_largest(sc, 0)
    corr = row_mask & col_mask & (sc > _CONF) & valid
    sel = jnp.where(corr, sc, 0.0)

    win_ref[...] = sel[:32, :, 0:1]

    iota_j = jax.lax.broadcasted_iota(jnp.int32, (n, n, B), 1)
    iota_i = jax.lax.broadcasted_iota(jnp.int32, (n, n, B), 0)
    iota_b = jax.lax.broadcasted_iota(jnp.int32, (n, n, B), 2)
    flat_base = (iota_b * n + iota_i) * n

    cur = sel
    cvs, cis = [], []
    for _ in range(_FINE_TOPK):
        vk = jnp.max(cur, axis=1, keepdims=True)
        jk = jnp.min(jnp.where(cur == vk, iota_j, n), axis=1,
                     keepdims=True)
        cvs.append(jnp.where(vk > 0.0, vk, -1.0))
        cis.append(jnp.min(jnp.where(cur == vk, flat_base + iota_j,
                                     jnp.int32(2 ** 30)), axis=1,
                           keepdims=True))
        cur = jnp.where(iota_j == jk, -1.0, cur)
    candv_ref[...] = jnp.concatenate(cvs, axis=1)
    candi_ref[...] = jnp.concatenate(cis, axis=1)


def _ot_fine(tgt_ckf, src_ckf, tgt_ckm, src_ckm, alpha):
    b = tgt_ckf.shape[0]
    n = _POINT_LIMIT
    s = pl.pallas_call(
        _sim_body,
        grid=(b // _B_SIM,),
        in_specs=[
            pl.BlockSpec((_B_SIM, n, _D_COARSE), lambda i: (i, 0, 0)),
            pl.BlockSpec((_B_SIM, n, _D_COARSE), lambda i: (i, 0, 0)),
        ],
        out_specs=pl.BlockSpec((_B_SIM, n, n), lambda i: (i, 0, 0)),
        out_shape=jax.ShapeDtypeStruct((b, n, n), jnp.float32),
    )(tgt_ckf, src_ckf)

    sT = jnp.transpose(s, (1, 2, 0))
    rmT = jnp.transpose(tgt_ckm.astype(jnp.float32))
    cmT = jnp.transpose(src_ckm.astype(jnp.float32))

    msT, candv, candi, win = pl.pallas_call(
        _ot_fine_body,
        in_specs=[
            pl.BlockSpec((n, n, b), lambda: (0, 0, 0)),
            pl.BlockSpec((n, b), lambda: (0, 0)),
            pl.BlockSpec((n, b), lambda: (0, 0)),
            pl.BlockSpec(memory_space=pltpu.SMEM),
        ],
        out_specs=[
            pl.BlockSpec((n + 1, n + 1, b), lambda: (0, 0, 0)),
            pl.BlockSpec((n, _FINE_TOPK, b), lambda: (0, 0, 0)),
            pl.BlockSpec((n, _FINE_TOPK, b), lambda: (0, 0, 0)),
            pl.BlockSpec((32, n, 1), lambda: (0, 0, 0)),
        ],
        out_shape=[
            jax.ShapeDtypeStruct((n + 1, n + 1, b), jnp.float32),
            jax.ShapeDtypeStruct((n, _FINE_TOPK, b), jnp.float32),
            jax.ShapeDtypeStruct((n, _FINE_TOPK, b), jnp.int32),
            jax.ShapeDtypeStruct((32, n, 1), jnp.float32),
        ],
    )(sT, rmT, cmT, alpha.reshape(1, 1))

    ms = jnp.transpose(msT, (2, 0, 1))
    return ms, candv, candi, win


def _fine_topk(candv, candi, win):
    n = _POINT_LIMIT
    cand_v = jnp.transpose(candv, (2, 0, 1)).reshape(-1)
    cand_i = jnp.transpose(candi, (2, 0, 1)).reshape(-1)
    w = win.reshape(-1)
    zkey = jnp.where(w == 0.0, -jnp.arange(w.shape[0], dtype=jnp.float32),
                     -3e9)
    _, zidx = jax.lax.top_k(zkey, _FINE_TOPK_TOTAL)
    allv = jnp.concatenate([cand_v, jnp.zeros((_FINE_TOPK_TOTAL,), jnp.float32)])
    alli = jnp.concatenate([cand_i, zidx.astype(jnp.int32)])
    vals, pos = jax.lax.top_k(allv, _FINE_TOPK_TOTAL)
    idx = alli[pos]
    return vals, idx


_N_ROWS_PAD = 320
_N_PTS = 20000
_NV = _N_PTS // 16


def _knn_sc_rows(dist2_padded):
    mesh = plsc.VectorSubcoreMesh(core_axis_name="c", subcore_axis_name="s")
    big_f = jnp.float32(3e38)
    big_i = jnp.int32(2 ** 30)

    @functools.partial(
        pl.kernel, mesh=mesh,
        compiler_params=pltpu.CompilerParams(needs_layout_passes=False),
        out_type=jax.ShapeDtypeStruct((_N_ROWS_PAD, 64), jnp.int32),
        scratch_types=[
            pltpu.VMEM((_N_PTS,), jnp.float32),
            pltpu.VMEM((_N_PTS + 16,), jnp.float32),
            pltpu.VMEM((_N_PTS + 16,), jnp.int32),
            pltpu.VMEM((64,), jnp.int32),
        ],
    )
    def knn_kernel(d_hbm, out_hbm, drow_v, candv_v, candi_v, outrow_v):
        wid = lax.axis_index("s") * 2 + lax.axis_index("c")
        iota = lax.iota(jnp.int32, 16)
        iota_f = iota.astype(jnp.float32)

        def do_row(t, _):
            rid = wid + 32 * t
            pltpu.sync_copy(d_hbm.at[rid], drow_v)

            def s1(j, carry):
                m1, m2, m3, m4 = carry
                v = plsc.load_gather(drow_v, [j * 16 + iota])
                t1 = jnp.maximum(m1, v)
                m1 = jnp.minimum(m1, v)
                t2 = jnp.maximum(m2, t1)
                m2 = jnp.minimum(m2, t1)
                t3 = jnp.maximum(m3, t2)
                m3 = jnp.minimum(m3, t2)
                m4 = jnp.minimum(m4, t3)
                return (m1, m2, m3, m4)

            inf16 = jnp.full((16,), big_f, jnp.float32)
            _, _, _, m4 = lax.fori_loop(0, _NV, s1, (inf16, inf16, inf16, inf16))
            tau = jnp.max(m4)
            tau_v = jnp.full((16,), tau, jnp.float32)

            def s2(j, off):
                idx = j * 16 + iota
                v = plsc.load_gather(drow_v, [idx])
                mask = v <= tau_v
                cs = plsc.cumsum(mask.astype(jnp.int32))
                pos = off + cs - 1
                plsc.store_scatter(candv_v, [pos], v, mask=mask)
                plsc.store_scatter(candi_v, [pos], idx, mask=mask)
                return off + plsc.all_reduce_population_count(mask)

            off = lax.fori_loop(0, _NV, s2, jnp.zeros((16,), jnp.int32))
            n_cand = jnp.max(off)
            plsc.store_scatter(candv_v, [n_cand + iota],
                               jnp.full((16,), big_f, jnp.float32))
            nv = (n_cand + 15) // 16

            def s3(r, _):
                def p1(i, m):
                    v = plsc.load_gather(candv_v, [i * 16 + iota])
                    return jnp.minimum(m, v)

                mvec = lax.fori_loop(0, nv, p1, inf16)
                m = jnp.min(mvec)
                m_v = jnp.full((16,), m, jnp.float32)

                def p2(i, fp):
                    v = plsc.load_gather(candv_v, [i * 16 + iota])
                    pm = jnp.where(v == m_v, i * 16 + iota, big_i)
                    return jnp.minimum(fp, pm)

                fpvec = lax.fori_loop(0, nv, p2,
                                      jnp.full((16,), big_i, jnp.int32))
                p = jnp.min(fpvec)
                p_v = jnp.full((16,), p, jnp.int32)
                oi = plsc.load_gather(candi_v, [p_v])
                lane0 = iota == 0
                plsc.store_scatter(outrow_v, [jnp.full((16,), r, jnp.int32)],
                                   oi, mask=lane0)
                plsc.store_scatter(candv_v, [p_v],
                                   jnp.full((16,), big_f, jnp.float32), mask=lane0)
                return 0

            lax.fori_loop(0, 64, s3, 0)
            pltpu.sync_copy(outrow_v, out_hbm.at[rid])
            return 0

        lax.fori_loop(0, _N_ROWS_PAD // 32, do_row, 0)

    return knn_kernel(dist2_padded)


def _backbone(points, normals, feats, v, Wn1, Wn2, Wp1):
    h = jnp.concatenate([points, normals, feats, v], axis=1)
    point_feats = jax.nn.relu(h @ Wp1)
    node_xyz = points[::_STRIDE]
    node_h = h[::_STRIDE]
    node_feats = jax.nn.relu(node_h @ Wn1) @ Wn2
    return node_xyz, node_feats, point_feats


def _point_to_node_partition(points, nodes, point_limit):
    dist2 = (jnp.sum(nodes ** 2, axis=1)[:, None]
             + jnp.sum(points ** 2, axis=1)[None, :]
             - 2.0 * (nodes @ points.T))
    point_to_node = jnp.argmin(dist2, axis=0)
    n_nodes = nodes.shape[0]
    counts = jnp.bincount(point_to_node, length=n_nodes)
    node_masks = counts > 0
    pad = jnp.broadcast_to(jnp.arange(_N_PTS, dtype=jnp.float32),
                           (_N_ROWS_PAD - n_nodes, _N_PTS))
    knn_indices = _knn_sc_rows(
        jnp.concatenate([dist2, pad], axis=0))[:n_nodes]
    knn_masks = point_to_node[knn_indices] == jnp.arange(n_nodes)[:, None]
    knn_indices = jnp.where(knn_masks, knn_indices, points.shape[0])
    return point_to_node, node_masks, knn_indices, knn_masks


def _coarse_matching(tgt_feats, src_feats, tgt_masks, src_masks, k):
    sim = jnp.exp(tgt_feats @ src_feats.T)
    row = sim / (jnp.sum(sim, axis=1, keepdims=True) + 1e-12)
    col = sim / (jnp.sum(sim, axis=0, keepdims=True) + 1e-12)
    score = row * col
    mask = tgt_masks[:, None] & src_masks[None, :]
    score = jnp.where(mask, score, 0.0)
    vals, idx = jax.lax.top_k(score.reshape(-1), k)
    m = src_feats.shape[0]
    return idx // m, idx % m, vals


def kernel(src_points, tgt_points, src_normals, tgt_normals, src_feats,
           tgt_feats, src_v, tgt_v, Wn1, Wn2, Wp1, W_coarse, b_coarse,
           W_fine, b_fine, alpha):
    src_node_xyz, src_node_feats, src_point_feats = _backbone(
        src_points, src_normals, src_feats, src_v, Wn1, Wn2, Wp1)
    tgt_node_xyz, tgt_node_feats, tgt_point_feats = _backbone(
        tgt_points, tgt_normals, tgt_feats, tgt_v, Wn1, Wn2, Wp1)

    def l2n(x):
        return x / (jnp.linalg.norm(x, axis=1, keepdims=True) + 1e-12)

    src_node_feats = l2n(src_node_feats @ W_coarse + b_coarse)
    tgt_node_feats = l2n(tgt_node_feats @ W_coarse + b_coarse)
    src_point_feats = src_point_feats @ W_fine + b_fine
    tgt_point_feats = tgt_point_feats @ W_fine + b_fine

    _, src_node_masks, src_knn_idx, src_knn_masks = _point_to_node_partition(
        src_points, src_node_xyz, _POINT_LIMIT)
    _, tgt_node_masks, tgt_knn_idx, tgt_knn_masks = _point_to_node_partition(
        tgt_points, tgt_node_xyz, _POINT_LIMIT)

    tgt_ci, src_ci, _ = _coarse_matching(
        tgt_node_feats, src_node_feats, tgt_node_masks, src_node_masks,
        _NUM_CORR)

    src_ckm = src_knn_masks[src_ci]
    tgt_ckm = tgt_knn_masks[tgt_ci]
    src_ckidx = src_knn_idx[src_ci]
    tgt_ckidx = tgt_knn_idx[tgt_ci]

    src_pp_feats = jnp.concatenate(
        [src_point_feats, jnp.zeros((1, _D_COARSE), jnp.float32)], axis=0)
    tgt_pp_feats = jnp.concatenate(
        [tgt_point_feats, jnp.zeros((1, _D_COARSE), jnp.float32)], axis=0)
    src_ckf = src_pp_feats[src_ckidx]
    tgt_ckf = tgt_pp_feats[tgt_ckidx]

    ms, candv, candi, win = _ot_fine(tgt_ckf, src_ckf, tgt_ckm, src_ckm, alpha)
    vals, idx = _fine_topk(candv, candi, win)

    n = m = _POINT_LIMIT
    bi = idx // (n * m)
    rem = idx % (n * m)

    src_padded_points = jnp.concatenate(
        [src_points, jnp.zeros((1, 3), src_points.dtype)], axis=0)
    tgt_padded_points = jnp.concatenate(
        [tgt_points, jnp.zeros((1, 3), tgt_points.dtype)], axis=0)
    tgt_corr_points = tgt_padded_points[tgt_ckidx[bi, rem // m]]
    src_corr_points = src_padded_points[src_ckidx[bi, rem % m]]
    return ms, tgt_corr_points, src_corr_points, vals
